# baseline (speedup 1.0000x reference)
import functools

import jax
import jax.numpy as jnp
from jax import lax
from jax.experimental import pallas as pl
from jax.experimental.pallas import tpu as pltpu
from jax.experimental.pallas import tpu_sc as plsc

N = 10000
E = 320000
D = 128
H = 128
C = 10
L = 3
G = 64
BN_EPS = 1e-5

NC = 2
NS = 16
NW = NC * NS
EPT = E // NW
K = 125
NCHUNK = EPT // K
NBUF = 2
NGROUP = NCHUNK // NBUF
RPT = 624
REM = N - NS * RPT
ZROWS = 16

BLK = 1000
NBLK = N // BLK


def _make_agg():
    mesh = plsc.VectorSubcoreMesh(core_axis_name="c", subcore_axis_name="s",
                                  num_cores=NC, num_subcores=NS)

    @functools.partial(
        pl.kernel,
        mesh=mesh,
        out_type=jax.ShapeDtypeStruct((NC * N, D), jnp.float32),
        scratch_types=[
            pltpu.VMEM((2, NBUF, 2, K), jnp.int32),
            pltpu.VMEM((NBUF, K, D), jnp.float32),
            pltpu.VMEM((ZROWS, D), jnp.float32),
            pltpu.VMEM_SHARED((N, D), jnp.float32),
            pltpu.SemaphoreType.DMA,
            pltpu.SemaphoreType.DMA,
            pltpu.SemaphoreType.DMA,
        ],
    )
    def agg(h_hbm, e_hbm, out_hbm, idx_v, rows_v, z_v, acc_sh, sem_g, sem_i,
            sem_s):
        c = lax.axis_index("c")
        s = lax.axis_index("s")
        wid = c * NS + s

        zero16 = jnp.zeros((16,), jnp.float32)
        for r in range(ZROWS):
            for kk in range(D // 16):
                z_v[r, pl.ds(kk * 16, 16)] = zero16

        row0 = s * RPT

        def zbody(i, carry):
            pltpu.sync_copy(z_v, acc_sh.at[pl.ds(row0 + i * ZROWS, ZROWS)])
            return carry

        lax.fori_loop(0, RPT // ZROWS, zbody, 0)

        @pl.when(s == NS - 1)
        def _():
            pltpu.sync_copy(z_v.at[pl.ds(0, REM)],
                            acc_sh.at[pl.ds(NS * RPT, REM)])

        for b in range(NBUF):
            pltpu.async_copy(e_hbm.at[wid, b], idx_v.at[0, b], sem_i)

        plsc.subcore_barrier()

        def gbody(g, carry):
            p = lax.rem(g, 2)
            q = lax.rem(g + 1, 2)

            @pl.when(g > 0)
            def _():
                for b in range(NBUF):
                    pltpu.make_async_copy(rows_v.at[b],
                                          acc_sh.at[idx_v.at[q, b, 1]],
                                          sem_s).wait()

            for b in range(NBUF):
                ci = g * NBUF + b
                pltpu.make_async_copy(e_hbm.at[wid, ci], idx_v.at[p, b],
                                      sem_i).wait()
                pltpu.async_copy(h_hbm.at[idx_v.at[p, b, 0]], rows_v.at[b],
                                 sem_g)
            for b in range(NBUF):
                ci = g * NBUF + b
                pltpu.make_async_copy(h_hbm.at[idx_v.at[p, b, 0]],
                                      rows_v.at[b], sem_g).wait()
                pltpu.async_copy(rows_v.at[b], acc_sh.at[idx_v.at[p, b, 1]],
                                 sem_s, add=True)

                @pl.when(ci + NBUF < NCHUNK)
                def _():
                    pltpu.async_copy(e_hbm.at[wid, ci + NBUF],
                                     idx_v.at[q, b], sem_i)
            return carry

        lax.fori_loop(0, NGROUP, gbody, 0)

        qf = (NGROUP - 1) % 2
        for b in range(NBUF):
            pltpu.make_async_copy(rows_v.at[b], acc_sh.at[idx_v.at[qf, b, 1]],
                                  sem_s).wait()

        plsc.subcore_barrier()
        off_out = pl.multiple_of(c * N + row0, 8)
        pltpu.sync_copy(acc_sh.at[pl.ds(row0, RPT)],
                        out_hbm.at[pl.ds(off_out, RPT)])

        @pl.when(s == NS - 1)
        def _():
            off_rem = pl.multiple_of(c * N + NS * RPT, 8)
            pltpu.sync_copy(acc_sh.at[pl.ds(NS * RPT, REM)],
                            out_hbm.at[pl.ds(off_rem, REM)])

    return agg


_agg = _make_agg()


def _mlp_body(h_ref, a0_ref, a1_ref, w1_ref, w2_ref, misc_ref, out_ref):
    z = h_ref[...] * misc_ref[4:5, :] + a0_ref[...] + a1_ref[...]
    z = jnp.dot(z, w1_ref[...], preferred_element_type=jnp.float32)
    z = jnp.maximum(z + misc_ref[0:1, :], 0.0)
    z = jnp.dot(z, w2_ref[...], preferred_element_type=jnp.float32)
    z = jnp.maximum(z + misc_ref[1:2, :], 0.0)
    out_ref[...] = z * misc_ref[2:3, :] + misc_ref[3:4, :]


def _mlp_call(h, agg, W1, W2, misc):
    return pl.pallas_call(
        _mlp_body,
        grid=(NBLK,),
        in_specs=[
            pl.BlockSpec((BLK, D), lambda i: (i, 0)),
            pl.BlockSpec((BLK, D), lambda i: (i, 0)),
            pl.BlockSpec((BLK, D), lambda i: (i + NBLK, 0)),
            pl.BlockSpec((D, H), lambda i: (0, 0)),
            pl.BlockSpec((H, H), lambda i: (0, 0)),
            pl.BlockSpec((8, H), lambda i: (0, 0)),
        ],
        out_specs=pl.BlockSpec((BLK, H), lambda i: (i, 0)),
        out_shape=jax.ShapeDtypeStruct((N, H), jnp.float32),
    )(h, agg, agg, W1, W2, misc)


def _final_body(h_ref, a0_ref, a1_ref, w1_ref, w2_ref, misc_ref, batch_ref,
                l1w_ref, l2w_ref, head_ref, out_ref, acc, cnt):
    i = pl.program_id(0)

    @pl.when(i == 0)
    def _():
        acc[...] = jnp.zeros_like(acc)
        cnt[...] = jnp.zeros_like(cnt)

    z = h_ref[...] * misc_ref[4:5, :] + a0_ref[...] + a1_ref[...]
    z = jnp.dot(z, w1_ref[...], preferred_element_type=jnp.float32)
    z = jnp.maximum(z + misc_ref[0:1, :], 0.0)
    z = jnp.dot(z, w2_ref[...], preferred_element_type=jnp.float32)
    z = jnp.maximum(z + misc_ref[1:2, :], 0.0)
    z = z * misc_ref[2:3, :] + misc_ref[3:4, :]

    seg = lax.broadcasted_iota(jnp.int32, (G, BLK), 0)
    ohT = (seg == batch_ref[0]).astype(jnp.float32)
    acc[...] += jnp.dot(ohT, z, preferred_element_type=jnp.float32)
    cnt[...] += jnp.dot(ohT, jnp.ones((BLK, H), jnp.float32),
                        preferred_element_type=jnp.float32)

    @pl.when(i == NBLK - 1)
    def _():
        pooled = acc[...] / jnp.maximum(cnt[...], 1.0)
        ph = jnp.dot(pooled, l1w_ref[...], preferred_element_type=jnp.float32)
        ph = jnp.maximum(ph + head_ref[0:1, :], 0.0)
        logits = jnp.dot(ph, l2w_ref[...], preferred_element_type=jnp.float32)
        logits = logits + head_ref[1:2, :]
        m = jnp.max(logits, axis=1, keepdims=True)
        lse = jnp.log(jnp.sum(jnp.exp(logits - m), axis=1, keepdims=True))
        out_ref[...] = logits - m - lse


def _final_call(h, agg, W1, W2, misc, batch3d, l1w, l2wp, headp):
    return pl.pallas_call(
        _final_body,
        grid=(NBLK,),
        in_specs=[
            pl.BlockSpec((BLK, D), lambda i: (i, 0)),
            pl.BlockSpec((BLK, D), lambda i: (i, 0)),
            pl.BlockSpec((BLK, D), lambda i: (i + NBLK, 0)),
            pl.BlockSpec((D, H), lambda i: (0, 0)),
            pl.BlockSpec((H, H), lambda i: (0, 0)),
            pl.BlockSpec((8, H), lambda i: (0, 0)),
            pl.BlockSpec((1, 1, BLK), lambda i: (i, 0, 0)),
            pl.BlockSpec((H, H), lambda i: (0, 0)),
            pl.BlockSpec((H, H), lambda i: (0, 0)),
            pl.BlockSpec((8, H), lambda i: (0, 0)),
        ],
        out_specs=pl.BlockSpec((G, H), lambda i: (0, 0)),
        out_shape=jax.ShapeDtypeStruct((G, H), jnp.float32),
        scratch_shapes=[
            pltpu.VMEM((G, H), jnp.float32),
            pltpu.VMEM((G, H), jnp.float32),
        ],
    )(h, agg, agg, W1, W2, misc, batch3d, l1w, l2wp, headp)


def kernel(x, edge_index, batch, Wa, ba, Wb, bb, gamma, beta, eps,
           lin1_W, lin1_b, lin2_W, lin2_b):
    e4 = edge_index.reshape(2, NW, NCHUNK, K).transpose(1, 2, 0, 3)
    batch3d = batch.reshape(NBLK, 1, BLK)
    inv_bn = 1.0 / (1.0 + BN_EPS) ** 0.5

    l2wp = jnp.concatenate(
        [lin2_W, jnp.zeros((H, H - C), jnp.float32)], axis=1)
    l2bp = jnp.concatenate(
        [lin2_b, jnp.full((H - C,), -1e30, jnp.float32)], axis=0)
    headp = jnp.concatenate(
        [lin1_b[None, :], l2bp[None, :], jnp.zeros((6, H), jnp.float32)], 0)

    h = x
    out = None
    for i in range(L):
        misc = jnp.stack([
            ba[i], bb[i], gamma[i] * inv_bn, beta[i],
            jnp.full((H,), 1.0, jnp.float32) + eps[i],
        ], 0)
        misc = jnp.concatenate([misc, jnp.zeros((3, H), jnp.float32)], 0)
        agg = _agg(h, e4)
        if i < L - 1:
            h = _mlp_call(h, agg, Wa[i], Wb[i], misc)
        else:
            out = _final_call(h, agg, Wa[i], Wb[i], misc, batch3d,
                              lin1_W, l2wp, headp)
    return out[:, :C]


# speedup vs baseline: 1.0583x; 1.0583x over previous
import functools

import jax
import jax.numpy as jnp
from jax import lax
from jax.experimental import pallas as pl
from jax.experimental.pallas import tpu as pltpu
from jax.experimental.pallas import tpu_sc as plsc

N = 10000
E = 320000
D = 128
H = 128
C = 10
L = 3
G = 64
BN_EPS = 1e-5

NC = 2
NS = 16
NW = NC * NS
EPT = E // NW
K = 50
NCHUNK = EPT // K
NBUF = 5
NGROUP = NCHUNK // NBUF
RPT = 624
REM = N - NS * RPT
ZROWS = 24
ZCOPIES = RPT // ZROWS
ZWAVE = ZCOPIES // 2

BLK = 2000
NBLK = N // BLK


def _make_agg():
    mesh = plsc.VectorSubcoreMesh(core_axis_name="c", subcore_axis_name="s",
                                  num_cores=NC, num_subcores=NS)

    @functools.partial(
        pl.kernel,
        mesh=mesh,
        out_type=jax.ShapeDtypeStruct((NC * N, D), jnp.float32),
        scratch_types=[
            pltpu.VMEM((2, NBUF, 2, K), jnp.int32),
            pltpu.VMEM((NBUF, K, D), jnp.float32),
            pltpu.VMEM((ZROWS, D), jnp.float32),
            pltpu.VMEM_SHARED((N, D), jnp.float32),
            pltpu.SemaphoreType.DMA,
            pltpu.SemaphoreType.DMA,
            pltpu.SemaphoreType.DMA,
        ],
    )
    def agg(h_hbm, e_hbm, out_hbm, idx_v, rows_v, z_v, acc_sh, sem_g, sem_i,
            sem_s):
        c = lax.axis_index("c")
        s = lax.axis_index("s")
        wid = c * NS + s

        zero16 = jnp.zeros((16,), jnp.float32)
        for r in range(ZROWS):
            for kk in range(D // 16):
                z_v[r, pl.ds(kk * 16, 16)] = zero16

        row0 = s * RPT

        def zwave(w, carry):
            def zfire(i, c):
                pltpu.async_copy(
                    z_v, acc_sh.at[pl.ds(row0 + (w * ZWAVE + i) * ZROWS,
                                         ZROWS)], sem_s)
                return c

            lax.fori_loop(0, ZWAVE, zfire, 0)

            def zdrain(i, c):
                pltpu.make_async_copy(
                    z_v, acc_sh.at[pl.ds(row0 + (w * ZWAVE + i) * ZROWS,
                                         ZROWS)], sem_s).wait()
                return c

            lax.fori_loop(0, ZWAVE, zdrain, 0)
            return carry

        lax.fori_loop(0, 2, zwave, 0)

        @pl.when(s == NS - 1)
        def _():
            pltpu.sync_copy(z_v.at[pl.ds(0, REM)],
                            acc_sh.at[pl.ds(NS * RPT, REM)])

        for b in range(NBUF):
            pltpu.async_copy(e_hbm.at[wid, b], idx_v.at[0, b], sem_i)

        plsc.subcore_barrier()

        def gbody(g, carry):
            p = lax.rem(g, 2)
            q = lax.rem(g + 1, 2)

            @pl.when(g > 0)
            def _():
                for b in range(NBUF):
                    pltpu.make_async_copy(rows_v.at[b],
                                          acc_sh.at[idx_v.at[q, b, 1]],
                                          sem_s).wait()

            for b in range(NBUF):
                ci = g * NBUF + b
                pltpu.make_async_copy(e_hbm.at[wid, ci], idx_v.at[p, b],
                                      sem_i).wait()
                pltpu.async_copy(h_hbm.at[idx_v.at[p, b, 0]], rows_v.at[b],
                                 sem_g)
            for b in range(NBUF):
                ci = g * NBUF + b
                pltpu.make_async_copy(h_hbm.at[idx_v.at[p, b, 0]],
                                      rows_v.at[b], sem_g).wait()
                pltpu.async_copy(rows_v.at[b], acc_sh.at[idx_v.at[p, b, 1]],
                                 sem_s, add=True)

                @pl.when(ci + NBUF < NCHUNK)
                def _():
                    pltpu.async_copy(e_hbm.at[wid, ci + NBUF],
                                     idx_v.at[q, b], sem_i)
            return carry

        lax.fori_loop(0, NGROUP, gbody, 0)

        qf = (NGROUP - 1) % 2
        for b in range(NBUF):
            pltpu.make_async_copy(rows_v.at[b], acc_sh.at[idx_v.at[qf, b, 1]],
                                  sem_s).wait()

        plsc.subcore_barrier()
        off_out = pl.multiple_of(c * N + row0, 8)
        pltpu.sync_copy(acc_sh.at[pl.ds(row0, RPT)],
                        out_hbm.at[pl.ds(off_out, RPT)])

        @pl.when(s == NS - 1)
        def _():
            off_rem = pl.multiple_of(c * N + NS * RPT, 8)
            pltpu.sync_copy(acc_sh.at[pl.ds(NS * RPT, REM)],
                            out_hbm.at[pl.ds(off_rem, REM)])

    return agg


_agg = _make_agg()


def _mlp_body(h_ref, a0_ref, a1_ref, w1_ref, w2_ref, misc_ref, out_ref):
    z = h_ref[...] * misc_ref[4:5, :] + a0_ref[...] + a1_ref[...]
    z = jnp.dot(z, w1_ref[...], preferred_element_type=jnp.float32)
    z = jnp.maximum(z + misc_ref[0:1, :], 0.0)
    z = jnp.dot(z, w2_ref[...], preferred_element_type=jnp.float32)
    z = jnp.maximum(z + misc_ref[1:2, :], 0.0)
    out_ref[...] = z * misc_ref[2:3, :] + misc_ref[3:4, :]


def _mlp_call(h, agg, W1, W2, misc):
    return pl.pallas_call(
        _mlp_body,
        grid=(NBLK,),
        in_specs=[
            pl.BlockSpec((BLK, D), lambda i: (i, 0)),
            pl.BlockSpec((BLK, D), lambda i: (i, 0)),
            pl.BlockSpec((BLK, D), lambda i: (i + NBLK, 0)),
            pl.BlockSpec((D, H), lambda i: (0, 0)),
            pl.BlockSpec((H, H), lambda i: (0, 0)),
            pl.BlockSpec((8, H), lambda i: (0, 0)),
        ],
        out_specs=pl.BlockSpec((BLK, H), lambda i: (i, 0)),
        out_shape=jax.ShapeDtypeStruct((N, H), jnp.float32),
    )(h, agg, agg, W1, W2, misc)


def _final_body(h_ref, a0_ref, a1_ref, w1_ref, w2_ref, misc_ref, batch_ref,
                l1w_ref, l2w_ref, head_ref, out_ref, acc, cnt):
    i = pl.program_id(0)

    @pl.when(i == 0)
    def _():
        acc[...] = jnp.zeros_like(acc)
        cnt[...] = jnp.zeros_like(cnt)

    z = h_ref[...] * misc_ref[4:5, :] + a0_ref[...] + a1_ref[...]
    z = jnp.dot(z, w1_ref[...], preferred_element_type=jnp.float32)
    z = jnp.maximum(z + misc_ref[0:1, :], 0.0)
    z = jnp.dot(z, w2_ref[...], preferred_element_type=jnp.float32)
    z = jnp.maximum(z + misc_ref[1:2, :], 0.0)
    z = z * misc_ref[2:3, :] + misc_ref[3:4, :]

    seg = lax.broadcasted_iota(jnp.int32, (G, BLK), 0)
    ohT = (seg == batch_ref[0]).astype(jnp.float32)
    acc[...] += jnp.dot(ohT, z, preferred_element_type=jnp.float32)
    cnt[...] += jnp.dot(ohT, jnp.ones((BLK, H), jnp.float32),
                        preferred_element_type=jnp.float32)

    @pl.when(i == NBLK - 1)
    def _():
        pooled = acc[...] / jnp.maximum(cnt[...], 1.0)
        ph = jnp.dot(pooled, l1w_ref[...], preferred_element_type=jnp.float32)
        ph = jnp.maximum(ph + head_ref[0:1, :], 0.0)
        logits = jnp.dot(ph, l2w_ref[...], preferred_element_type=jnp.float32)
        logits = logits + head_ref[1:2, :]
        m = jnp.max(logits, axis=1, keepdims=True)
        lse = jnp.log(jnp.sum(jnp.exp(logits - m), axis=1, keepdims=True))
        out_ref[...] = logits - m - lse


def _final_call(h, agg, W1, W2, misc, batch3d, l1w, l2wp, headp):
    return pl.pallas_call(
        _final_body,
        grid=(NBLK,),
        in_specs=[
            pl.BlockSpec((BLK, D), lambda i: (i, 0)),
            pl.BlockSpec((BLK, D), lambda i: (i, 0)),
            pl.BlockSpec((BLK, D), lambda i: (i + NBLK, 0)),
            pl.BlockSpec((D, H), lambda i: (0, 0)),
            pl.BlockSpec((H, H), lambda i: (0, 0)),
            pl.BlockSpec((8, H), lambda i: (0, 0)),
            pl.BlockSpec((1, 1, BLK), lambda i: (i, 0, 0)),
            pl.BlockSpec((H, H), lambda i: (0, 0)),
            pl.BlockSpec((H, H), lambda i: (0, 0)),
            pl.BlockSpec((8, H), lambda i: (0, 0)),
        ],
        out_specs=pl.BlockSpec((G, H), lambda i: (0, 0)),
        out_shape=jax.ShapeDtypeStruct((G, H), jnp.float32),
        scratch_shapes=[
            pltpu.VMEM((G, H), jnp.float32),
            pltpu.VMEM((G, H), jnp.float32),
        ],
    )(h, agg, agg, W1, W2, misc, batch3d, l1w, l2wp, headp)


def kernel(x, edge_index, batch, Wa, ba, Wb, bb, gamma, beta, eps,
           lin1_W, lin1_b, lin2_W, lin2_b):
    e4 = edge_index.reshape(2, NW, NCHUNK, K).transpose(1, 2, 0, 3)
    batch3d = batch.reshape(NBLK, 1, BLK)
    inv_bn = 1.0 / (1.0 + BN_EPS) ** 0.5

    l2wp = jnp.concatenate(
        [lin2_W, jnp.zeros((H, H - C), jnp.float32)], axis=1)
    l2bp = jnp.concatenate(
        [lin2_b, jnp.full((H - C,), -1e30, jnp.float32)], axis=0)
    headp = jnp.concatenate(
        [lin1_b[None, :], l2bp[None, :], jnp.zeros((6, H), jnp.float32)], 0)

    h = x
    out = None
    for i in range(L):
        misc = jnp.stack([
            ba[i], bb[i], gamma[i] * inv_bn, beta[i],
            jnp.full((H,), 1.0, jnp.float32) + eps[i],
        ], 0)
        misc = jnp.concatenate([misc, jnp.zeros((3, H), jnp.float32)], 0)
        agg = _agg(h, e4)
        if i < L - 1:
            h = _mlp_call(h, agg, Wa[i], Wb[i], misc)
        else:
            out = _final_call(h, agg, Wa[i], Wb[i], misc, batch3d,
                              lin1_W, l2wp, headp)
    return out[:, :C]


# speedup vs baseline: 1.2721x; 1.2021x over previous
import functools

import jax
import jax.numpy as jnp
from jax import lax
from jax.experimental import pallas as pl
from jax.experimental.pallas import tpu as pltpu
from jax.experimental.pallas import tpu_sc as plsc

N = 10000
E = 320000
D = 128
H = 128
C = 10
L = 3
G = 64
BN_EPS = 1e-5

NC = 2
NS = 16
NW = NC * NS
EPT = E // NW
K = 50
NCHUNK = EPT // K
NBUF = 5
NGROUP = NCHUNK // NBUF
SKEW = 3
RPT = 624
REM = N - NS * RPT
ZROWS = 24
ZCOPIES = RPT // ZROWS
ZWAVE = ZCOPIES // 2

BLK = 2000
NBLK = N // BLK


def _make_agg():
    mesh = plsc.VectorSubcoreMesh(core_axis_name="c", subcore_axis_name="s",
                                  num_cores=NC, num_subcores=NS)

    @functools.partial(
        pl.kernel,
        mesh=mesh,
        out_type=jax.ShapeDtypeStruct((NC * N, D), jnp.float32),
        scratch_types=[
            pltpu.VMEM((2, NBUF, 2, K), jnp.int32),
            pltpu.VMEM((NBUF, K, D), jnp.float32),
            pltpu.VMEM((ZROWS, D), jnp.float32),
            pltpu.VMEM_SHARED((N, D), jnp.float32),
            pltpu.SemaphoreType.DMA((NBUF,)),
            pltpu.SemaphoreType.DMA((2, NBUF)),
            pltpu.SemaphoreType.DMA((NBUF,)),
            pltpu.SemaphoreType.DMA,
        ],
    )
    def agg(h_hbm, e_hbm, out_hbm, idx_v, rows_v, z_v, acc_sh, sem_g, sem_i,
            sem_s, sem_z):
        c = lax.axis_index("c")
        s = lax.axis_index("s")
        wid = c * NS + s

        zero16 = jnp.zeros((16,), jnp.float32)
        for r in range(ZROWS):
            for kk in range(D // 16):
                z_v[r, pl.ds(kk * 16, 16)] = zero16

        row0 = s * RPT

        def zwave(w, carry):
            def zfire(i, c):
                pltpu.async_copy(
                    z_v, acc_sh.at[pl.ds(row0 + (w * ZWAVE + i) * ZROWS,
                                         ZROWS)], sem_z)
                return c

            lax.fori_loop(0, ZWAVE, zfire, 0)

            def zdrain(i, c):
                pltpu.make_async_copy(
                    z_v, acc_sh.at[pl.ds(row0 + (w * ZWAVE + i) * ZROWS,
                                         ZROWS)], sem_z).wait()
                return c

            lax.fori_loop(0, ZWAVE, zdrain, 0)
            return carry

        lax.fori_loop(0, 2, zwave, 0)

        @pl.when(s == NS - 1)
        def _():
            pltpu.sync_copy(z_v.at[pl.ds(0, REM)],
                            acc_sh.at[pl.ds(NS * RPT, REM)])

        for b in range(NBUF):
            pltpu.async_copy(e_hbm.at[wid, b], idx_v.at[0, b], sem_i.at[0, b])

        plsc.subcore_barrier()

        def body(i, carry):
            b = lax.rem(i, NBUF)
            gi = i // NBUF
            p = lax.rem(gi, 2)
            q = lax.rem(gi + 1, 2)

            @pl.when(i >= NBUF)
            def _():
                pltpu.make_async_copy(rows_v.at[b],
                                      acc_sh.at[idx_v.at[q, b, 1]],
                                      sem_s.at[b]).wait()

            @pl.when(i + NBUF < NCHUNK)
            def _():
                pltpu.async_copy(e_hbm.at[wid, i + NBUF], idx_v.at[q, b],
                                 sem_i.at[q, b])

            pltpu.make_async_copy(e_hbm.at[wid, i], idx_v.at[p, b],
                                  sem_i.at[p, b]).wait()
            pltpu.async_copy(h_hbm.at[idx_v.at[p, b, 0]], rows_v.at[b],
                             sem_g.at[b])

            j = i - SKEW

            @pl.when(j >= 0)
            def _():
                bj = lax.rem(j, NBUF)
                pj = lax.rem(j // NBUF, 2)
                pltpu.make_async_copy(h_hbm.at[idx_v.at[pj, bj, 0]],
                                      rows_v.at[bj], sem_g.at[bj]).wait()
                pltpu.async_copy(rows_v.at[bj], acc_sh.at[idx_v.at[pj, bj, 1]],
                                 sem_s.at[bj], add=True)
            return carry

        lax.fori_loop(0, NCHUNK, body, 0)

        def ebody(j, carry):
            bj = lax.rem(j, NBUF)
            pj = lax.rem(j // NBUF, 2)
            pltpu.make_async_copy(h_hbm.at[idx_v.at[pj, bj, 0]],
                                  rows_v.at[bj], sem_g.at[bj]).wait()
            pltpu.async_copy(rows_v.at[bj], acc_sh.at[idx_v.at[pj, bj, 1]],
                             sem_s.at[bj], add=True)
            return carry

        lax.fori_loop(NCHUNK - SKEW, NCHUNK, ebody, 0)

        qf = (NGROUP - 1) % 2
        for b in range(NBUF):
            pltpu.make_async_copy(rows_v.at[b], acc_sh.at[idx_v.at[qf, b, 1]],
                                  sem_s.at[b]).wait()

        plsc.subcore_barrier()
        off_out = pl.multiple_of(c * N + row0, 8)
        pltpu.sync_copy(acc_sh.at[pl.ds(row0, RPT)],
                        out_hbm.at[pl.ds(off_out, RPT)])

        @pl.when(s == NS - 1)
        def _():
            off_rem = pl.multiple_of(c * N + NS * RPT, 8)
            pltpu.sync_copy(acc_sh.at[pl.ds(NS * RPT, REM)],
                            out_hbm.at[pl.ds(off_rem, REM)])

    return agg


_agg = _make_agg()


def _mlp_body(h_ref, a0_ref, a1_ref, w1_ref, w2_ref, misc_ref, out_ref):
    z = h_ref[...] * misc_ref[4:5, :] + a0_ref[...] + a1_ref[...]
    z = jnp.dot(z, w1_ref[...], preferred_element_type=jnp.float32)
    z = jnp.maximum(z + misc_ref[0:1, :], 0.0)
    z = jnp.dot(z, w2_ref[...], preferred_element_type=jnp.float32)
    z = jnp.maximum(z + misc_ref[1:2, :], 0.0)
    out_ref[...] = z * misc_ref[2:3, :] + misc_ref[3:4, :]


def _mlp_call(h, agg, W1, W2, misc):
    return pl.pallas_call(
        _mlp_body,
        grid=(NBLK,),
        in_specs=[
            pl.BlockSpec((BLK, D), lambda i: (i, 0)),
            pl.BlockSpec((BLK, D), lambda i: (i, 0)),
            pl.BlockSpec((BLK, D), lambda i: (i + NBLK, 0)),
            pl.BlockSpec((D, H), lambda i: (0, 0)),
            pl.BlockSpec((H, H), lambda i: (0, 0)),
            pl.BlockSpec((8, H), lambda i: (0, 0)),
        ],
        out_specs=pl.BlockSpec((BLK, H), lambda i: (i, 0)),
        out_shape=jax.ShapeDtypeStruct((N, H), jnp.float32),
    )(h, agg, agg, W1, W2, misc)


def _final_body(h_ref, a0_ref, a1_ref, w1_ref, w2_ref, misc_ref, batch_ref,
                l1w_ref, l2w_ref, head_ref, out_ref, acc, cnt):
    i = pl.program_id(0)

    @pl.when(i == 0)
    def _():
        acc[...] = jnp.zeros_like(acc)
        cnt[...] = jnp.zeros_like(cnt)

    z = h_ref[...] * misc_ref[4:5, :] + a0_ref[...] + a1_ref[...]
    z = jnp.dot(z, w1_ref[...], preferred_element_type=jnp.float32)
    z = jnp.maximum(z + misc_ref[0:1, :], 0.0)
    z = jnp.dot(z, w2_ref[...], preferred_element_type=jnp.float32)
    z = jnp.maximum(z + misc_ref[1:2, :], 0.0)
    z = z * misc_ref[2:3, :] + misc_ref[3:4, :]

    seg = lax.broadcasted_iota(jnp.int32, (G, BLK), 0)
    ohT = (seg == batch_ref[0]).astype(jnp.float32)
    acc[...] += jnp.dot(ohT, z, preferred_element_type=jnp.float32)
    cnt[...] += jnp.dot(ohT, jnp.ones((BLK, H), jnp.float32),
                        preferred_element_type=jnp.float32)

    @pl.when(i == NBLK - 1)
    def _():
        pooled = acc[...] / jnp.maximum(cnt[...], 1.0)
        ph = jnp.dot(pooled, l1w_ref[...], preferred_element_type=jnp.float32)
        ph = jnp.maximum(ph + head_ref[0:1, :], 0.0)
        logits = jnp.dot(ph, l2w_ref[...], preferred_element_type=jnp.float32)
        logits = logits + head_ref[1:2, :]
        m = jnp.max(logits, axis=1, keepdims=True)
        lse = jnp.log(jnp.sum(jnp.exp(logits - m), axis=1, keepdims=True))
        out_ref[...] = logits - m - lse


def _final_call(h, agg, W1, W2, misc, batch3d, l1w, l2wp, headp):
    return pl.pallas_call(
        _final_body,
        grid=(NBLK,),
        in_specs=[
            pl.BlockSpec((BLK, D), lambda i: (i, 0)),
            pl.BlockSpec((BLK, D), lambda i: (i, 0)),
            pl.BlockSpec((BLK, D), lambda i: (i + NBLK, 0)),
            pl.BlockSpec((D, H), lambda i: (0, 0)),
            pl.BlockSpec((H, H), lambda i: (0, 0)),
            pl.BlockSpec((8, H), lambda i: (0, 0)),
            pl.BlockSpec((1, 1, BLK), lambda i: (i, 0, 0)),
            pl.BlockSpec((H, H), lambda i: (0, 0)),
            pl.BlockSpec((H, H), lambda i: (0, 0)),
            pl.BlockSpec((8, H), lambda i: (0, 0)),
        ],
        out_specs=pl.BlockSpec((G, H), lambda i: (0, 0)),
        out_shape=jax.ShapeDtypeStruct((G, H), jnp.float32),
        scratch_shapes=[
            pltpu.VMEM((G, H), jnp.float32),
            pltpu.VMEM((G, H), jnp.float32),
        ],
    )(h, agg, agg, W1, W2, misc, batch3d, l1w, l2wp, headp)


def kernel(x, edge_index, batch, Wa, ba, Wb, bb, gamma, beta, eps,
           lin1_W, lin1_b, lin2_W, lin2_b):
    e4 = edge_index.reshape(2, NW, NCHUNK, K).transpose(1, 2, 0, 3)
    batch3d = batch.reshape(NBLK, 1, BLK)
    inv_bn = 1.0 / (1.0 + BN_EPS) ** 0.5

    l2wp = jnp.concatenate(
        [lin2_W, jnp.zeros((H, H - C), jnp.float32)], axis=1)
    l2bp = jnp.concatenate(
        [lin2_b, jnp.full((H - C,), -1e30, jnp.float32)], axis=0)
    headp = jnp.concatenate(
        [lin1_b[None, :], l2bp[None, :], jnp.zeros((6, H), jnp.float32)], 0)

    h = x
    out = None
    for i in range(L):
        misc = jnp.stack([
            ba[i], bb[i], gamma[i] * inv_bn, beta[i],
            jnp.full((H,), 1.0, jnp.float32) + eps[i],
        ], 0)
        misc = jnp.concatenate([misc, jnp.zeros((3, H), jnp.float32)], 0)
        agg = _agg(h, e4)
        if i < L - 1:
            h = _mlp_call(h, agg, Wa[i], Wb[i], misc)
        else:
            out = _final_call(h, agg, Wa[i], Wb[i], misc, batch3d,
                              lin1_W, l2wp, headp)
    return out[:, :C]


# speedup vs baseline: 1.2725x; 1.0003x over previous
import functools

import jax
import jax.numpy as jnp
from jax import lax
from jax.experimental import pallas as pl
from jax.experimental.pallas import tpu as pltpu
from jax.experimental.pallas import tpu_sc as plsc

N = 10000
E = 320000
D = 128
H = 128
C = 10
L = 3
G = 64
BN_EPS = 1e-5

NC = 2
NS = 16
NW = NC * NS
EPT = E // NW
K = 50
NCHUNK = EPT // K
NBUF = 5
NGROUP = NCHUNK // NBUF
SKEW = 3
RPT = 624
REM = N - NS * RPT
ZROWS = 24
ZCOPIES = RPT // ZROWS
ZWAVE = ZCOPIES // 2

BLK = 2000
NBLK = N // BLK


def _make_agg():
    mesh = plsc.VectorSubcoreMesh(core_axis_name="c", subcore_axis_name="s",
                                  num_cores=NC, num_subcores=NS)

    @functools.partial(
        pl.kernel,
        mesh=mesh,
        out_type=jax.ShapeDtypeStruct((NC * N, D), jnp.float32),
        scratch_types=[
            pltpu.VMEM((2, NBUF, 2, K), jnp.int32),
            pltpu.VMEM((NBUF, K, D), jnp.float32),
            pltpu.VMEM((ZROWS, D), jnp.float32),
            pltpu.VMEM_SHARED((N, D), jnp.float32),
            pltpu.SemaphoreType.DMA((NBUF,)),
            pltpu.SemaphoreType.DMA((2, NBUF)),
            pltpu.SemaphoreType.DMA((NBUF,)),
            pltpu.SemaphoreType.DMA,
        ],
    )
    def agg(h_hbm, e_hbm, out_hbm, idx_v, rows_v, z_v, acc_sh, sem_g, sem_i,
            sem_s, sem_z):
        c = lax.axis_index("c")
        s = lax.axis_index("s")
        wid = c * NS + s

        for b in range(NBUF):
            pltpu.async_copy(e_hbm.at[wid, b], idx_v.at[0, b], sem_i.at[0, b])

        zero16 = jnp.zeros((16,), jnp.float32)
        for r in range(ZROWS):
            for kk in range(D // 16):
                z_v[r, pl.ds(kk * 16, 16)] = zero16

        row0 = s * RPT

        def zwave(w, carry):
            def zfire(i, c):
                pltpu.async_copy(
                    z_v, acc_sh.at[pl.ds(row0 + (w * ZWAVE + i) * ZROWS,
                                         ZROWS)], sem_z)
                return c

            lax.fori_loop(0, ZWAVE, zfire, 0)

            def zdrain(i, c):
                pltpu.make_async_copy(
                    z_v, acc_sh.at[pl.ds(row0 + (w * ZWAVE + i) * ZROWS,
                                         ZROWS)], sem_z).wait()
                return c

            lax.fori_loop(0, ZWAVE, zdrain, 0)
            return carry

        lax.fori_loop(0, 2, zwave, 0)

        @pl.when(s == NS - 1)
        def _():
            pltpu.sync_copy(z_v.at[pl.ds(0, REM)],
                            acc_sh.at[pl.ds(NS * RPT, REM)])

        plsc.subcore_barrier()

        def gbody(g, carry):
            p = lax.rem(g, 2)
            q = lax.rem(g + 1, 2)
            base = g * NBUF
            for b in range(NBUF):
                i = base + b

                @pl.when(g > 0)
                def _():
                    pltpu.make_async_copy(rows_v.at[b],
                                          acc_sh.at[idx_v.at[q, b, 1]],
                                          sem_s.at[b]).wait()

                @pl.when(i + NBUF < NCHUNK)
                def _():
                    pltpu.async_copy(e_hbm.at[wid, i + NBUF], idx_v.at[q, b],
                                     sem_i.at[q, b])

                pltpu.make_async_copy(e_hbm.at[wid, i], idx_v.at[p, b],
                                      sem_i.at[p, b]).wait()
                pltpu.async_copy(h_hbm.at[idx_v.at[p, b, 0]], rows_v.at[b],
                                 sem_g.at[b])

                bj = (b - SKEW) % NBUF
                pj = p if b >= SKEW else q

                def _scatter(bj=bj, pj=pj):
                    pltpu.make_async_copy(h_hbm.at[idx_v.at[pj, bj, 0]],
                                          rows_v.at[bj], sem_g.at[bj]).wait()
                    pltpu.async_copy(rows_v.at[bj],
                                     acc_sh.at[idx_v.at[pj, bj, 1]],
                                     sem_s.at[bj], add=True)

                if b >= SKEW:
                    _scatter()
                else:
                    pl.when(g > 0)(_scatter)
            return carry

        lax.fori_loop(0, NGROUP, gbody, 0)

        qf = (NGROUP - 1) % 2
        for t in range(SKEW):
            bj = NBUF - SKEW + t
            pltpu.make_async_copy(h_hbm.at[idx_v.at[qf, bj, 0]],
                                  rows_v.at[bj], sem_g.at[bj]).wait()
            pltpu.async_copy(rows_v.at[bj], acc_sh.at[idx_v.at[qf, bj, 1]],
                             sem_s.at[bj], add=True)

        for b in range(NBUF):
            pltpu.make_async_copy(rows_v.at[b], acc_sh.at[idx_v.at[qf, b, 1]],
                                  sem_s.at[b]).wait()

        plsc.subcore_barrier()
        off_out = pl.multiple_of(c * N + row0, 8)
        pltpu.sync_copy(acc_sh.at[pl.ds(row0, RPT)],
                        out_hbm.at[pl.ds(off_out, RPT)])

        @pl.when(s == NS - 1)
        def _():
            off_rem = pl.multiple_of(c * N + NS * RPT, 8)
            pltpu.sync_copy(acc_sh.at[pl.ds(NS * RPT, REM)],
                            out_hbm.at[pl.ds(off_rem, REM)])

    return agg


_agg = _make_agg()


def _mlp_body(h_ref, a0_ref, a1_ref, w1_ref, w2_ref, misc_ref, out_ref):
    z = h_ref[...] * misc_ref[4:5, :] + a0_ref[...] + a1_ref[...]
    z = jnp.dot(z, w1_ref[...], preferred_element_type=jnp.float32)
    z = jnp.maximum(z + misc_ref[0:1, :], 0.0)
    z = jnp.dot(z, w2_ref[...], preferred_element_type=jnp.float32)
    z = jnp.maximum(z + misc_ref[1:2, :], 0.0)
    out_ref[...] = z * misc_ref[2:3, :] + misc_ref[3:4, :]


def _mlp_call(h, agg, W1, W2, misc):
    return pl.pallas_call(
        _mlp_body,
        grid=(NBLK,),
        in_specs=[
            pl.BlockSpec((BLK, D), lambda i: (i, 0)),
            pl.BlockSpec((BLK, D), lambda i: (i, 0)),
            pl.BlockSpec((BLK, D), lambda i: (i + NBLK, 0)),
            pl.BlockSpec((D, H), lambda i: (0, 0)),
            pl.BlockSpec((H, H), lambda i: (0, 0)),
            pl.BlockSpec((8, H), lambda i: (0, 0)),
        ],
        out_specs=pl.BlockSpec((BLK, H), lambda i: (i, 0)),
        out_shape=jax.ShapeDtypeStruct((N, H), jnp.float32),
    )(h, agg, agg, W1, W2, misc)


def _final_body(h_ref, a0_ref, a1_ref, w1_ref, w2_ref, misc_ref, batch_ref,
                l1w_ref, l2w_ref, head_ref, out_ref, acc, cnt):
    i = pl.program_id(0)

    @pl.when(i == 0)
    def _():
        acc[...] = jnp.zeros_like(acc)
        cnt[...] = jnp.zeros_like(cnt)

    z = h_ref[...] * misc_ref[4:5, :] + a0_ref[...] + a1_ref[...]
    z = jnp.dot(z, w1_ref[...], preferred_element_type=jnp.float32)
    z = jnp.maximum(z + misc_ref[0:1, :], 0.0)
    z = jnp.dot(z, w2_ref[...], preferred_element_type=jnp.float32)
    z = jnp.maximum(z + misc_ref[1:2, :], 0.0)
    z = z * misc_ref[2:3, :] + misc_ref[3:4, :]

    seg = lax.broadcasted_iota(jnp.int32, (G, BLK), 0)
    ohT = (seg == batch_ref[0]).astype(jnp.float32)
    acc[...] += jnp.dot(ohT, z, preferred_element_type=jnp.float32)
    cnt[...] += jnp.dot(ohT, jnp.ones((BLK, H), jnp.float32),
                        preferred_element_type=jnp.float32)

    @pl.when(i == NBLK - 1)
    def _():
        pooled = acc[...] / jnp.maximum(cnt[...], 1.0)
        ph = jnp.dot(pooled, l1w_ref[...], preferred_element_type=jnp.float32)
        ph = jnp.maximum(ph + head_ref[0:1, :], 0.0)
        logits = jnp.dot(ph, l2w_ref[...], preferred_element_type=jnp.float32)
        logits = logits + head_ref[1:2, :]
        m = jnp.max(logits, axis=1, keepdims=True)
        lse = jnp.log(jnp.sum(jnp.exp(logits - m), axis=1, keepdims=True))
        out_ref[...] = logits - m - lse


def _final_call(h, agg, W1, W2, misc, batch3d, l1w, l2wp, headp):
    return pl.pallas_call(
        _final_body,
        grid=(NBLK,),
        in_specs=[
            pl.BlockSpec((BLK, D), lambda i: (i, 0)),
            pl.BlockSpec((BLK, D), lambda i: (i, 0)),
            pl.BlockSpec((BLK, D), lambda i: (i + NBLK, 0)),
            pl.BlockSpec((D, H), lambda i: (0, 0)),
            pl.BlockSpec((H, H), lambda i: (0, 0)),
            pl.BlockSpec((8, H), lambda i: (0, 0)),
            pl.BlockSpec((1, 1, BLK), lambda i: (i, 0, 0)),
            pl.BlockSpec((H, H), lambda i: (0, 0)),
            pl.BlockSpec((H, H), lambda i: (0, 0)),
            pl.BlockSpec((8, H), lambda i: (0, 0)),
        ],
        out_specs=pl.BlockSpec((G, H), lambda i: (0, 0)),
        out_shape=jax.ShapeDtypeStruct((G, H), jnp.float32),
        scratch_shapes=[
            pltpu.VMEM((G, H), jnp.float32),
            pltpu.VMEM((G, H), jnp.float32),
        ],
    )(h, agg, agg, W1, W2, misc, batch3d, l1w, l2wp, headp)


def kernel(x, edge_index, batch, Wa, ba, Wb, bb, gamma, beta, eps,
           lin1_W, lin1_b, lin2_W, lin2_b):
    e4 = edge_index.reshape(2, NW, NCHUNK, K).transpose(1, 2, 0, 3)
    batch3d = batch.reshape(NBLK, 1, BLK)
    inv_bn = 1.0 / (1.0 + BN_EPS) ** 0.5

    l2wp = jnp.concatenate(
        [lin2_W, jnp.zeros((H, H - C), jnp.float32)], axis=1)
    l2bp = jnp.concatenate(
        [lin2_b, jnp.full((H - C,), -1e30, jnp.float32)], axis=0)
    headp = jnp.concatenate(
        [lin1_b[None, :], l2bp[None, :], jnp.zeros((6, H), jnp.float32)], 0)

    h = x
    out = None
    for i in range(L):
        misc = jnp.stack([
            ba[i], bb[i], gamma[i] * inv_bn, beta[i],
            jnp.full((H,), 1.0, jnp.float32) + eps[i],
        ], 0)
        misc = jnp.concatenate([misc, jnp.zeros((3, H), jnp.float32)], 0)
        agg = _agg(h, e4)
        if i < L - 1:
            h = _mlp_call(h, agg, Wa[i], Wb[i], misc)
        else:
            out = _final_call(h, agg, Wa[i], Wb[i], misc, batch3d,
                              lin1_W, l2wp, headp)
    return out[:, :C]


# speedup vs baseline: 1.2742x; 1.0013x over previous
import functools

import jax
import jax.numpy as jnp
from jax import lax
from jax.experimental import pallas as pl
from jax.experimental.pallas import tpu as pltpu
from jax.experimental.pallas import tpu_sc as plsc

N = 10000
E = 320000
D = 128
H = 128
C = 10
L = 3
G = 64
BN_EPS = 1e-5

NC = 2
NS = 16
NW = NC * NS
EPT = E // NW
K = 50
NCHUNK = EPT // K
NBUF = 5
NGROUP = NCHUNK // NBUF
SKEW = 3
RPT = 624
REM = N - NS * RPT
ZROWS = 24
ZCOPIES = RPT // ZROWS
ZWAVE = ZCOPIES // 2

BLK = 2000
NBLK = N // BLK


def _make_agg():
    mesh = plsc.VectorSubcoreMesh(core_axis_name="c", subcore_axis_name="s",
                                  num_cores=NC, num_subcores=NS)

    @functools.partial(
        pl.kernel,
        mesh=mesh,
        out_type=jax.ShapeDtypeStruct((NC * N, D), jnp.float32),
        scratch_types=[
            pltpu.VMEM((2, NBUF, 2, K), jnp.int32),
            pltpu.VMEM((NBUF, K, D), jnp.float32),
            pltpu.VMEM((ZROWS, D), jnp.float32),
            pltpu.VMEM_SHARED((N, D), jnp.float32),
            pltpu.SemaphoreType.DMA((NBUF,)),
            pltpu.SemaphoreType.DMA((2, NBUF)),
            pltpu.SemaphoreType.DMA((NBUF,)),
            pltpu.SemaphoreType.DMA,
        ],
    )
    def agg(h_hbm, e_hbm, out_hbm, idx_v, rows_v, z_v, acc_sh, sem_g, sem_i,
            sem_s, sem_z):
        c = lax.axis_index("c")
        s = lax.axis_index("s")
        wid = c * NS + s

        for b in range(NBUF):
            pltpu.async_copy(e_hbm.at[wid, b], idx_v.at[0, b], sem_i.at[0, b])

        zero16 = jnp.zeros((16,), jnp.float32)
        for r in range(ZROWS):
            for kk in range(D // 16):
                z_v[r, pl.ds(kk * 16, 16)] = zero16

        row0 = s * RPT

        def zwave(w, carry):
            def zfire(i, c):
                pltpu.async_copy(
                    z_v, acc_sh.at[pl.ds(row0 + (w * ZWAVE + i) * ZROWS,
                                         ZROWS)], sem_z)
                return c

            lax.fori_loop(0, ZWAVE, zfire, 0)

            def zdrain(i, c):
                pltpu.make_async_copy(
                    z_v, acc_sh.at[pl.ds(row0 + (w * ZWAVE + i) * ZROWS,
                                         ZROWS)], sem_z).wait()
                return c

            lax.fori_loop(0, ZWAVE, zdrain, 0)
            return carry

        lax.fori_loop(0, 2, zwave, 0)

        @pl.when(s == NS - 1)
        def _():
            pltpu.sync_copy(z_v.at[pl.ds(0, REM)],
                            acc_sh.at[pl.ds(NS * RPT, REM)])

        plsc.subcore_barrier()

        def gbody(g, carry):
            p = lax.rem(g, 2)
            q = lax.rem(g + 1, 2)
            base = g * NBUF
            for b in range(NBUF):
                i = base + b

                @pl.when(g > 0)
                def _():
                    pltpu.make_async_copy(rows_v.at[b],
                                          acc_sh.at[idx_v.at[q, b, 1]],
                                          sem_s.at[b]).wait()

                @pl.when(i + NBUF < NCHUNK)
                def _():
                    pltpu.async_copy(e_hbm.at[wid, i + NBUF], idx_v.at[q, b],
                                     sem_i.at[q, b])

                pltpu.make_async_copy(e_hbm.at[wid, i], idx_v.at[p, b],
                                      sem_i.at[p, b]).wait()
                pltpu.async_copy(h_hbm.at[idx_v.at[p, b, 0]], rows_v.at[b],
                                 sem_g.at[b])

                bj = (b - SKEW) % NBUF
                pj = p if b >= SKEW else q

                def _scatter(bj=bj, pj=pj):
                    pltpu.make_async_copy(h_hbm.at[idx_v.at[pj, bj, 0]],
                                          rows_v.at[bj], sem_g.at[bj]).wait()
                    pltpu.async_copy(rows_v.at[bj],
                                     acc_sh.at[idx_v.at[pj, bj, 1]],
                                     sem_s.at[bj], add=True)

                if b >= SKEW:
                    _scatter()
                else:
                    pl.when(g > 0)(_scatter)
            return carry

        lax.fori_loop(0, NGROUP, gbody, 0)

        qf = (NGROUP - 1) % 2
        for t in range(SKEW):
            bj = NBUF - SKEW + t
            pltpu.make_async_copy(h_hbm.at[idx_v.at[qf, bj, 0]],
                                  rows_v.at[bj], sem_g.at[bj]).wait()
            pltpu.async_copy(rows_v.at[bj], acc_sh.at[idx_v.at[qf, bj, 1]],
                             sem_s.at[bj], add=True)

        for b in range(NBUF):
            pltpu.make_async_copy(rows_v.at[b], acc_sh.at[idx_v.at[qf, b, 1]],
                                  sem_s.at[b]).wait()

        plsc.subcore_barrier()
        off_out = pl.multiple_of(c * N + row0, 8)
        pltpu.sync_copy(acc_sh.at[pl.ds(row0, RPT)],
                        out_hbm.at[pl.ds(off_out, RPT)])

        @pl.when(s == NS - 1)
        def _():
            off_rem = pl.multiple_of(c * N + NS * RPT, 8)
            pltpu.sync_copy(acc_sh.at[pl.ds(NS * RPT, REM)],
                            out_hbm.at[pl.ds(off_rem, REM)])

    return agg


_agg = _make_agg()


_INV_BN = 1.0 / (1.0 + BN_EPS) ** 0.5


def _mlp_core(h_ref, a0_ref, a1_ref, w1_ref, w2_ref, b1_ref, b2_ref,
              g_ref, bt_ref, eps_ref):
    z = h_ref[...] * (1.0 + eps_ref[0, 0, 0]) + a0_ref[...] + a1_ref[...]
    z = jnp.dot(z, w1_ref[0], preferred_element_type=jnp.float32)
    z = jnp.maximum(z + b1_ref[0], 0.0)
    z = jnp.dot(z, w2_ref[0], preferred_element_type=jnp.float32)
    z = jnp.maximum(z + b2_ref[0], 0.0)
    return z * (g_ref[0] * _INV_BN) + bt_ref[0]


def _mlp_body(h_ref, a0_ref, a1_ref, w1_ref, w2_ref, b1_ref, b2_ref,
              g_ref, bt_ref, eps_ref, out_ref):
    out_ref[...] = _mlp_core(h_ref, a0_ref, a1_ref, w1_ref, w2_ref,
                             b1_ref, b2_ref, g_ref, bt_ref, eps_ref)


def _layer_specs(layer):
    return [
        pl.BlockSpec((BLK, D), lambda i: (i, 0)),
        pl.BlockSpec((BLK, D), lambda i: (i, 0)),
        pl.BlockSpec((BLK, D), lambda i: (i + NBLK, 0)),
        pl.BlockSpec((1, D, H), lambda i: (layer, 0, 0)),
        pl.BlockSpec((1, H, H), lambda i: (layer, 0, 0)),
        pl.BlockSpec((1, 1, H), lambda i: (layer, 0, 0)),
        pl.BlockSpec((1, 1, H), lambda i: (layer, 0, 0)),
        pl.BlockSpec((1, 1, H), lambda i: (layer, 0, 0)),
        pl.BlockSpec((1, 1, H), lambda i: (layer, 0, 0)),
        pl.BlockSpec((1, 1, 1), lambda i: (layer, 0, 0),
                     memory_space=pltpu.SMEM),
    ]


def _mlp_call(layer, h, agg, Wa, Wb, ba, bb, gamma, beta, eps2):
    return pl.pallas_call(
        _mlp_body,
        grid=(NBLK,),
        in_specs=_layer_specs(layer),
        out_specs=pl.BlockSpec((BLK, H), lambda i: (i, 0)),
        out_shape=jax.ShapeDtypeStruct((N, H), jnp.float32),
    )(h, agg, agg, Wa, Wb, ba, bb, gamma, beta, eps2)


def _final_body(h_ref, a0_ref, a1_ref, w1_ref, w2_ref, b1_ref, b2_ref,
                g_ref, bt_ref, eps_ref, batch_ref,
                l1w_ref, l2w_ref, head_ref, out_ref, acc, cnt):
    i = pl.program_id(0)

    @pl.when(i == 0)
    def _():
        acc[...] = jnp.zeros_like(acc)
        cnt[...] = jnp.zeros_like(cnt)

    z = _mlp_core(h_ref, a0_ref, a1_ref, w1_ref, w2_ref, b1_ref, b2_ref,
                  g_ref, bt_ref, eps_ref)

    seg = lax.broadcasted_iota(jnp.int32, (G, BLK), 0)
    ohT = (seg == batch_ref[0]).astype(jnp.float32)
    acc[...] += jnp.dot(ohT, z, preferred_element_type=jnp.float32)
    cnt[...] += jnp.dot(ohT, jnp.ones((BLK, H), jnp.float32),
                        preferred_element_type=jnp.float32)

    @pl.when(i == NBLK - 1)
    def _():
        pooled = acc[...] / jnp.maximum(cnt[...], 1.0)
        ph = jnp.dot(pooled, l1w_ref[...], preferred_element_type=jnp.float32)
        ph = jnp.maximum(ph + head_ref[0:1, :], 0.0)
        logits = jnp.dot(ph, l2w_ref[...], preferred_element_type=jnp.float32)
        logits = logits + head_ref[1:2, :]
        m = jnp.max(logits, axis=1, keepdims=True)
        lse = jnp.log(jnp.sum(jnp.exp(logits - m), axis=1, keepdims=True))
        out_ref[...] = logits - m - lse


def _final_call(layer, h, agg, Wa, Wb, ba, bb, gamma, beta, eps2,
                batch3d, l1w, l2wp, headp):
    return pl.pallas_call(
        _final_body,
        grid=(NBLK,),
        in_specs=_layer_specs(layer) + [
            pl.BlockSpec((1, 1, BLK), lambda i: (i, 0, 0)),
            pl.BlockSpec((H, H), lambda i: (0, 0)),
            pl.BlockSpec((H, H), lambda i: (0, 0)),
            pl.BlockSpec((8, H), lambda i: (0, 0)),
        ],
        out_specs=pl.BlockSpec((G, H), lambda i: (0, 0)),
        out_shape=jax.ShapeDtypeStruct((G, H), jnp.float32),
        scratch_shapes=[
            pltpu.VMEM((G, H), jnp.float32),
            pltpu.VMEM((G, H), jnp.float32),
        ],
    )(h, agg, agg, Wa, Wb, ba, bb, gamma, beta, eps2, batch3d, l1w, l2wp,
      headp)


def kernel(x, edge_index, batch, Wa, ba, Wb, bb, gamma, beta, eps,
           lin1_W, lin1_b, lin2_W, lin2_b):
    e4 = edge_index.reshape(2, NW, NCHUNK, K).transpose(1, 2, 0, 3)
    batch3d = batch.reshape(NBLK, 1, BLK)
    eps2 = eps.reshape(L, 1, 1)
    ba3, bb3 = ba.reshape(L, 1, H), bb.reshape(L, 1, H)
    gamma3, beta3 = gamma.reshape(L, 1, H), beta.reshape(L, 1, H)

    l2wp = jnp.concatenate(
        [lin2_W, jnp.zeros((H, H - C), jnp.float32)], axis=1)
    l2bp = jnp.concatenate(
        [lin2_b, jnp.full((H - C,), -1e30, jnp.float32)], axis=0)
    headp = jnp.concatenate(
        [lin1_b[None, :], l2bp[None, :], jnp.zeros((6, H), jnp.float32)], 0)

    h = x
    out = None
    for i in range(L):
        agg = _agg(h, e4)
        if i < L - 1:
            h = _mlp_call(i, h, agg, Wa, Wb, ba3, bb3, gamma3, beta3, eps2)
        else:
            out = _final_call(i, h, agg, Wa, Wb, ba3, bb3, gamma3, beta3,
                              eps2, batch3d, lin1_W, l2wp, headp)
    return out[:, :C]
